# baseline (speedup 1.0000x reference)
import functools

import jax
import jax.numpy as jnp
from jax import lax
from jax.experimental import pallas as pl
from jax.experimental.pallas import tpu as pltpu
from jax.experimental.pallas import tpu_sc as plsc

T = 2048
D = 2048
E = 8
W = 1024
K = 2
P = T * K
BLK = 128
S = 5120
NB = S // BLK

NC = 2
NS = 16
NW = NC * NS


def _router_body(x_ref, wr_ref, a_ref, cw_ref, st_ref, blke_ref, z_ref,
                 xbf_ref):
    x = x_ref[...]
    wr = wr_ref[...]
    xbf_ref[...] = x.astype(jnp.bfloat16)
    logits = lax.dot_general(x, wr, (((1,), (0,)), ((), ())),
                             preferred_element_type=jnp.float32)
    eidx = lax.broadcasted_iota(jnp.int32, (T, E), 1)
    m1 = jnp.max(logits, axis=1, keepdims=True)
    a1 = jnp.min(jnp.where(logits == m1, eidx, E), axis=1, keepdims=True)
    logits2 = jnp.where(eidx == a1, -jnp.inf, logits)
    m2 = jnp.max(logits2, axis=1, keepdims=True)
    a2 = jnp.min(jnp.where(logits2 == m2, eidx, E), axis=1, keepdims=True)
    s1 = jax.nn.sigmoid(m1)
    s2 = jax.nn.sigmoid(m2)
    denom = s1 + s2 + 1e-20
    a_ref[...] = jnp.concatenate([a1, a2], axis=1)
    cw_ref[...] = jnp.concatenate([s1 / denom, s2 / denom], axis=1)
    ohsum = (eidx == a1).astype(jnp.float32) + (eidx == a2).astype(jnp.float32)
    cnt = jnp.sum(ohsum, axis=0, keepdims=True)
    pc = jnp.ceil(cnt / BLK) * BLK
    ei = lax.broadcasted_iota(jnp.int32, (E, E), 0)
    ej = lax.broadcasted_iota(jnp.int32, (E, E), 1)
    strict = (ei < ej).astype(jnp.float32)
    starts = lax.dot_general(pc, strict, (((1,), (0,)), ((), ())),
                             preferred_element_type=jnp.float32)
    st_ref[...] = starts
    bi = lax.broadcasted_iota(jnp.int32, (NB, E), 0).astype(jnp.float32) * BLK
    blke_ref[...] = (jnp.sum((bi >= starts).astype(jnp.int32), axis=1,
                             keepdims=True) - 1)
    mx = jnp.max(logits, axis=1, keepdims=True)
    lse = mx + jnp.log(jnp.sum(jnp.exp(logits - mx), axis=1, keepdims=True))
    z_ref[...] = jnp.mean(lse * lse).reshape(1, 1)


def _router_call(x_flat, W_router):
    return pl.pallas_call(
        _router_body,
        out_shape=(
            jax.ShapeDtypeStruct((T, K), jnp.int32),
            jax.ShapeDtypeStruct((T, K), jnp.float32),
            jax.ShapeDtypeStruct((1, E), jnp.float32),
            jax.ShapeDtypeStruct((NB, 1), jnp.int32),
            jax.ShapeDtypeStruct((1, 1), jnp.float32),
            jax.ShapeDtypeStruct((T, D), jnp.bfloat16),
        ),
    )(x_flat, W_router)


_RB = 256


def _pos_body(a_ref, ab_ref, st_ref, pos_ref):
    i = pl.program_id(0)
    a1 = a_ref[:, 0:1]
    a2 = a_ref[:, 1:2]
    eidx = lax.broadcasted_iota(jnp.int32, (T, E), 1)
    amat = ((eidx == a1).astype(jnp.float32)
            + (eidx == a2).astype(jnp.float32))
    gi = lax.broadcasted_iota(jnp.int32, (_RB, T), 0) + i * _RB
    gj = lax.broadcasted_iota(jnp.int32, (_RB, T), 1)
    lmask = (gj < gi).astype(jnp.bfloat16)
    cex = lax.dot_general(lmask, amat.astype(jnp.bfloat16),
                          (((1,), (0,)), ((), ())),
                          preferred_element_type=jnp.float32)
    st = st_ref[...]
    eb = lax.broadcasted_iota(jnp.int32, (_RB, E), 1)
    a1b = ab_ref[:, 0:1]
    a2b = ab_ref[:, 1:2]
    oh1 = eb == a1b
    oh2 = eb == a2b
    pos1 = jnp.sum(jnp.where(oh1, cex + st, 0.0), axis=1, keepdims=True)
    pos2 = jnp.sum(jnp.where(oh2, cex + st, 0.0), axis=1, keepdims=True)
    pos_ref[...] = jnp.concatenate(
        [pos1.astype(jnp.int32), pos2.astype(jnp.int32)], axis=1)


def _pos_call(a1a2, starts):
    return pl.pallas_call(
        _pos_body,
        grid=(T // _RB,),
        in_specs=[
            pl.BlockSpec((T, K), lambda i: (0, 0)),
            pl.BlockSpec((_RB, K), lambda i: (i, 0)),
            pl.BlockSpec((1, E), lambda i: (0, 0)),
        ],
        out_specs=pl.BlockSpec((_RB, K), lambda i: (i, 0)),
        out_shape=jax.ShapeDtypeStruct((T, K), jnp.int32),
    )(a1a2, a1a2, starts)


def _slots_body(posr_ref, cwr_ref, tok_ref, cw_ref):
    j = pl.program_id(0)
    posr = posr_ref[...]
    cwr = cwr_ref[...]
    sid = lax.broadcasted_iota(jnp.int32, (BLK, P), 0) + j * BLK
    pidx = lax.broadcasted_iota(jnp.int32, (BLK, P), 1)
    m = sid == posr
    tok_ref[...] = jnp.sum(jnp.where(m, pidx >> 1, 0), axis=1, keepdims=True)
    cw_ref[...] = jnp.sum(jnp.where(m, cwr, 0.0), axis=1, keepdims=True)


def _slots_call(pos_row, cw_row):
    return pl.pallas_call(
        _slots_body,
        grid=(NB,),
        in_specs=[
            pl.BlockSpec((1, P), lambda j: (0, 0)),
            pl.BlockSpec((1, P), lambda j: (0, 0)),
        ],
        out_specs=(
            pl.BlockSpec((BLK, 1), lambda j: (j, 0)),
            pl.BlockSpec((BLK, 1), lambda j: (j, 0)),
        ),
        out_shape=(
            jax.ShapeDtypeStruct((S, 1), jnp.int32),
            jax.ShapeDtypeStruct((S, 1), jnp.float32),
        ),
    )(pos_row, cw_row)


_CROWS = 40
_RPW = S // NW
_NCH = _RPW // _CROWS


def _gather_body(x_hbm, idx_hbm, out_hbm, idx_v, b0, b1, sg0, sg1, sw0, sw1):
    wid = lax.axis_index("s") * NC + lax.axis_index("c")
    base = wid * _RPW
    pltpu.sync_copy(idx_hbm.at[pl.ds(base, _RPW)], idx_v)
    bufs, gsems, wsems = [b0, b1], [sg0, sg1], [sw0, sw1]

    def gath(c):
        return pltpu.async_copy(
            x_hbm.at[idx_v.at[pl.ds(c * _CROWS, _CROWS)]],
            bufs[c % 2], gsems[c % 2])

    def wr(c):
        return pltpu.async_copy(
            bufs[c % 2], out_hbm.at[pl.ds(base + c * _CROWS, _CROWS)],
            wsems[c % 2])

    gd = [None] * _NCH
    wd = [None] * _NCH
    gd[0] = gath(0)
    for c in range(_NCH):
        gd[c].wait()
        wd[c] = wr(c)
        if c + 1 < _NCH:
            if c >= 1:
                wd[c - 1].wait()
            gd[c + 1] = gath(c + 1)
    if _NCH >= 2:
        wd[_NCH - 2].wait()
    wd[_NCH - 1].wait()


def _gather_rows(x_bf, slot_tok):
    mesh = plsc.VectorSubcoreMesh(core_axis_name="c", subcore_axis_name="s",
                                  num_cores=NC, num_subcores=NS)
    fn = pl.kernel(
        _gather_body,
        out_type=jax.ShapeDtypeStruct((S, D // 2), jnp.int32),
        mesh=mesh,
        scratch_types=[
            pltpu.VMEM((_RPW,), jnp.int32),
            pltpu.VMEM((_CROWS, D // 2), jnp.int32),
            pltpu.VMEM((_CROWS, D // 2), jnp.int32),
            pltpu.SemaphoreType.DMA,
            pltpu.SemaphoreType.DMA,
            pltpu.SemaphoreType.DMA,
            pltpu.SemaphoreType.DMA,
        ],
    )
    return fn(x_bf, slot_tok)


def _mlp_body(blke_ref, xs_ref, w1_ref, w2_ref, cw_ref, out_ref):
    h = lax.dot_general(xs_ref[...].astype(jnp.float32), w1_ref[...],
                        (((1,), (0,)), ((), ())),
                        preferred_element_type=jnp.float32)
    h = jnp.square(jnp.maximum(h, 0.0))
    o = lax.dot_general(h, w2_ref[...], (((1,), (0,)), ((), ())),
                        preferred_element_type=jnp.float32)
    out_ref[...] = o * cw_ref[...]


def _mlp_call(blke, xs, w1, w2, slot_cw):
    grid_spec = pltpu.PrefetchScalarGridSpec(
        num_scalar_prefetch=1,
        grid=(NB,),
        in_specs=[
            pl.BlockSpec((BLK, D), lambda b, blke: (b, 0)),
            pl.BlockSpec((D, W), lambda b, blke: (0, blke[b])),
            pl.BlockSpec((W, D), lambda b, blke: (blke[b], 0)),
            pl.BlockSpec((BLK, 1), lambda b, blke: (b, 0)),
        ],
        out_specs=pl.BlockSpec((BLK, D), lambda b, blke: (b, 0)),
    )
    return pl.pallas_call(
        _mlp_body,
        grid_spec=grid_spec,
        out_shape=jax.ShapeDtypeStruct((S, D), jnp.float32),
    )(blke, xs, w1, w2, slot_cw)


_CTOK = 16
_TPW = T // NW
_NCHC = _TPW // _CTOK


def _combine_body(part_hbm, p0_hbm, p1_hbm, out_hbm, p0_v, p1_v, acc_v, g_v,
                  sem):
    wid = lax.axis_index("s") * NC + lax.axis_index("c")
    base = wid * _TPW
    pltpu.sync_copy(p0_hbm.at[pl.ds(base, _TPW)], p0_v)
    pltpu.sync_copy(p1_hbm.at[pl.ds(base, _TPW)], p1_v)

    def chunk(c, carry):
        d0 = pltpu.async_copy(
            part_hbm.at[p0_v.at[pl.ds(c * _CTOK, _CTOK)]], acc_v, sem)
        d1 = pltpu.async_copy(
            part_hbm.at[p1_v.at[pl.ds(c * _CTOK, _CTOK)]], g_v, sem)
        d0.wait()
        d1.wait()
        for r in range(_CTOK):
            def add16(ci, carry2, r=r):
                sl = pl.ds(ci * 16, 16)
                acc_v[r, sl] = acc_v[r, sl] + g_v[r, sl]
                return carry2
            lax.fori_loop(0, D // 16, add16, 0, unroll=8)
        pltpu.sync_copy(acc_v, out_hbm.at[pl.ds(base + c * _CTOK, _CTOK)])
        return carry

    lax.fori_loop(0, _NCHC, chunk, 0)


def _combine_rows(partial, p0, p1):
    mesh = plsc.VectorSubcoreMesh(core_axis_name="c", subcore_axis_name="s",
                                  num_cores=NC, num_subcores=NS)
    fn = pl.kernel(
        _combine_body,
        out_type=jax.ShapeDtypeStruct((T, D), jnp.float32),
        mesh=mesh,
        scratch_types=[
            pltpu.VMEM((_TPW,), jnp.int32),
            pltpu.VMEM((_TPW,), jnp.int32),
            pltpu.VMEM((_CTOK, D), jnp.float32),
            pltpu.VMEM((_CTOK, D), jnp.float32),
            pltpu.SemaphoreType.DMA,
        ],
    )
    return fn(partial, p0, p1)


def kernel(x, W_router, w1, w2):
    b, s, d = x.shape
    x_flat = x.reshape(T, D)
    a1a2, cw01, starts, blke, z, x_bf = _router_call(x_flat, W_router)
    pos01 = _pos_call(a1a2, starts)
    slot_tok_col, slot_cw = _slots_call(pos01.reshape(1, P),
                                        cw01.reshape(1, P))
    slot_tok = slot_tok_col.reshape(S)
    x_bfi = lax.bitcast_convert_type(x_bf.reshape(T, D // 2, 2), jnp.int32)
    xs_i = _gather_rows(x_bfi, slot_tok)
    xs = lax.bitcast_convert_type(xs_i, jnp.bfloat16).reshape(S, D)
    partial = _mlp_call(blke.reshape(NB), xs, w1, w2, slot_cw)
    out_flat = _combine_rows(partial, pos01[:, 0], pos01[:, 1])
    return out_flat.reshape(b, s, d), z.reshape(())


# speedup vs baseline: 2.0245x; 2.0245x over previous
import jax
import jax.numpy as jnp
from jax import lax
from jax.experimental import pallas as pl
from jax.experimental.pallas import tpu as pltpu
from jax.experimental.pallas import tpu_sc as plsc

T = 2048
D = 2048
E = 8
W = 1024
K = 2
BLK = 128
S = 5120
NB = S // BLK

NC = 2
NS = 16
NW = NC * NS

_HD = D // 2


def _bf16_bits(xf32):
    t = lax.bitcast_convert_type(xf32, jnp.int32)
    carry = jnp.bitwise_and(lax.shift_right_logical(t, 16), 1)
    return lax.shift_right_logical(t + 32767 + carry, 16)


def _router_body(x_ref, wr_ref, p0_ref, p1_ref, cw0_ref, cw1_ref,
                 blke_ref, z_ref, xp_ref):
    x = x_ref[...]
    wr = wr_ref[...]
    lo = _bf16_bits(x[:, :_HD])
    hi = _bf16_bits(x[:, _HD:])
    xp_ref[...] = jnp.bitwise_or(lo, lax.shift_left(hi, 16))
    logits = lax.dot_general(x, wr, (((1,), (0,)), ((), ())),
                             preferred_element_type=jnp.float32)
    eidx = lax.broadcasted_iota(jnp.int32, (T, E), 1)
    m1 = jnp.max(logits, axis=1, keepdims=True)
    a1 = jnp.min(jnp.where(logits == m1, eidx, E), axis=1, keepdims=True)
    logits2 = jnp.where(eidx == a1, -jnp.inf, logits)
    m2 = jnp.max(logits2, axis=1, keepdims=True)
    a2 = jnp.min(jnp.where(logits2 == m2, eidx, E), axis=1, keepdims=True)
    s1 = jax.nn.sigmoid(m1)
    s2 = jax.nn.sigmoid(m2)
    denom = s1 + s2 + 1e-20
    cw0_ref[...] = s1 / denom
    cw1_ref[...] = s2 / denom
    oh1 = eidx == a1
    oh2 = eidx == a2
    ohsum = oh1.astype(jnp.float32) + oh2.astype(jnp.float32)
    cnt = jnp.sum(ohsum, axis=0, keepdims=True)
    pc = jnp.ceil(cnt / BLK) * BLK
    ei = lax.broadcasted_iota(jnp.int32, (E, E), 0)
    ej = lax.broadcasted_iota(jnp.int32, (E, E), 1)
    strict = (ei < ej).astype(jnp.float32)
    starts = lax.dot_general(pc, strict, (((1,), (0,)), ((), ())),
                             preferred_element_type=jnp.float32)
    ti = lax.broadcasted_iota(jnp.int32, (T, 1), 0)
    tj = lax.broadcasted_iota(jnp.int32, (1, T), 1)
    lmask = (tj < ti).astype(jnp.bfloat16)
    cex = lax.dot_general(lmask, ohsum.astype(jnp.bfloat16),
                          (((1,), (0,)), ((), ())),
                          preferred_element_type=jnp.float32)
    pos_base = cex + starts
    p0_ref[...] = jnp.sum(jnp.where(oh1, pos_base, 0.0), axis=1,
                          keepdims=True).astype(jnp.int32)
    p1_ref[...] = jnp.sum(jnp.where(oh2, pos_base, 0.0), axis=1,
                          keepdims=True).astype(jnp.int32)
    bi = lax.broadcasted_iota(jnp.int32, (NB, E), 0).astype(jnp.float32) * BLK
    blke_ref[...] = (jnp.sum((bi >= starts).astype(jnp.int32), axis=1,
                             keepdims=True) - 1)
    mx = jnp.max(logits, axis=1, keepdims=True)
    lse = mx + jnp.log(jnp.sum(jnp.exp(logits - mx), axis=1, keepdims=True))
    z_ref[...] = jnp.mean(lse * lse).reshape(1, 1)


def _router_call(x_flat, W_router):
    return pl.pallas_call(
        _router_body,
        out_shape=(
            jax.ShapeDtypeStruct((T, 1), jnp.int32),
            jax.ShapeDtypeStruct((T, 1), jnp.int32),
            jax.ShapeDtypeStruct((T, 1), jnp.float32),
            jax.ShapeDtypeStruct((T, 1), jnp.float32),
            jax.ShapeDtypeStruct((NB, 1), jnp.int32),
            jax.ShapeDtypeStruct((1, 1), jnp.float32),
            jax.ShapeDtypeStruct((T, _HD), jnp.int32),
        ),
    )(x_flat, W_router)


def _slots_body(p0_ref, p1_ref, cw0_ref, cw1_ref, tok_ref, cw_ref):
    j = pl.program_id(0)
    p0r = p0_ref[...]
    p1r = p1_ref[...]
    cw0r = cw0_ref[...]
    cw1r = cw1_ref[...]
    sid = lax.broadcasted_iota(jnp.int32, (BLK, T), 0) + j * BLK
    tok = lax.broadcasted_iota(jnp.int32, (BLK, T), 1)
    m0 = sid == p0r
    m1 = sid == p1r
    tok_ref[...] = (jnp.sum(jnp.where(m0, tok, 0), axis=1, keepdims=True)
                    + jnp.sum(jnp.where(m1, tok, 0), axis=1, keepdims=True))
    cw_ref[...] = (jnp.sum(jnp.where(m0, cw0r, 0.0), axis=1, keepdims=True)
                   + jnp.sum(jnp.where(m1, cw1r, 0.0), axis=1, keepdims=True))


def _slots_call(p0_row, p1_row, cw0_row, cw1_row):
    return pl.pallas_call(
        _slots_body,
        grid=(NB,),
        in_specs=[pl.BlockSpec((1, T), lambda j: (0, 0))] * 4,
        out_specs=(
            pl.BlockSpec((BLK, 1), lambda j: (j, 0)),
            pl.BlockSpec((BLK, 1), lambda j: (j, 0)),
        ),
        out_shape=(
            jax.ShapeDtypeStruct((S, 1), jnp.int32),
            jax.ShapeDtypeStruct((S, 1), jnp.float32),
        ),
    )(p0_row, p1_row, cw0_row, cw1_row)


_CROWS = 40
_RPW = S // NW
_NCH = _RPW // _CROWS


def _gather_body(x_hbm, idx_hbm, out_hbm, idx_v, b0, b1, sg0, sg1, sw0, sw1):
    wid = lax.axis_index("s") * NC + lax.axis_index("c")
    base = wid * _RPW
    pltpu.sync_copy(idx_hbm.at[pl.ds(base, _RPW)], idx_v)
    bufs, gsems, wsems = [b0, b1], [sg0, sg1], [sw0, sw1]

    def gath(c):
        return pltpu.async_copy(
            x_hbm.at[idx_v.at[pl.ds(c * _CROWS, _CROWS)]],
            bufs[c % 2], gsems[c % 2])

    def wr(c):
        return pltpu.async_copy(
            bufs[c % 2], out_hbm.at[pl.ds(base + c * _CROWS, _CROWS)],
            wsems[c % 2])

    gd = [None] * _NCH
    wd = [None] * _NCH
    gd[0] = gath(0)
    for c in range(_NCH):
        gd[c].wait()
        wd[c] = wr(c)
        if c + 1 < _NCH:
            if c >= 1:
                wd[c - 1].wait()
            gd[c + 1] = gath(c + 1)
    if _NCH >= 2:
        wd[_NCH - 2].wait()
    wd[_NCH - 1].wait()


def _gather_rows(x_packed, slot_tok):
    mesh = plsc.VectorSubcoreMesh(core_axis_name="c", subcore_axis_name="s",
                                  num_cores=NC, num_subcores=NS)
    fn = pl.kernel(
        _gather_body,
        out_type=jax.ShapeDtypeStruct((S, _HD), jnp.int32),
        mesh=mesh,
        scratch_types=[
            pltpu.VMEM((_RPW,), jnp.int32),
            pltpu.VMEM((_CROWS, _HD), jnp.int32),
            pltpu.VMEM((_CROWS, _HD), jnp.int32),
            pltpu.SemaphoreType.DMA,
            pltpu.SemaphoreType.DMA,
            pltpu.SemaphoreType.DMA,
            pltpu.SemaphoreType.DMA,
        ],
    )
    return fn(x_packed, slot_tok)


def _mlp_body(blke_ref, xs_ref, w1_ref, w2_ref, cw_ref, out_ref):
    xp = xs_ref[...]
    xlo = lax.bitcast_convert_type(lax.shift_left(xp, 16), jnp.float32)
    xhi = lax.bitcast_convert_type(
        jnp.bitwise_and(xp, jnp.int32(-65536)), jnp.float32)
    xs = jnp.concatenate([xlo, xhi], axis=1)
    h = lax.dot_general(xs, w1_ref[...], (((1,), (0,)), ((), ())),
                        preferred_element_type=jnp.float32)
    h = jnp.square(jnp.maximum(h, 0.0))
    o = lax.dot_general(h, w2_ref[...], (((1,), (0,)), ((), ())),
                        preferred_element_type=jnp.float32)
    out_ref[...] = o * cw_ref[...]


def _mlp_call(blke, xs_packed, w1, w2, slot_cw):
    grid_spec = pltpu.PrefetchScalarGridSpec(
        num_scalar_prefetch=1,
        grid=(NB,),
        in_specs=[
            pl.BlockSpec((BLK, _HD), lambda b, blke: (b, 0)),
            pl.BlockSpec((D, W), lambda b, blke: (0, blke[b])),
            pl.BlockSpec((W, D), lambda b, blke: (blke[b], 0)),
            pl.BlockSpec((BLK, 1), lambda b, blke: (b, 0)),
        ],
        out_specs=pl.BlockSpec((BLK, D), lambda b, blke: (b, 0)),
    )
    return pl.pallas_call(
        _mlp_body,
        grid_spec=grid_spec,
        out_shape=jax.ShapeDtypeStruct((S, D), jnp.float32),
    )(blke, xs_packed, w1, w2, slot_cw)


_CTOK = 16
_TPW = T // NW
_NCHC = _TPW // _CTOK


def _combine_body(part_hbm, p0_hbm, p1_hbm, out_hbm, p0_v, p1_v, acc_v, g_v,
                  sem):
    wid = lax.axis_index("s") * NC + lax.axis_index("c")
    base = wid * _TPW
    pltpu.sync_copy(p0_hbm.at[pl.ds(base, _TPW)], p0_v)
    pltpu.sync_copy(p1_hbm.at[pl.ds(base, _TPW)], p1_v)

    def chunk(c, carry):
        d0 = pltpu.async_copy(
            part_hbm.at[p0_v.at[pl.ds(c * _CTOK, _CTOK)]], acc_v, sem)
        d1 = pltpu.async_copy(
            part_hbm.at[p1_v.at[pl.ds(c * _CTOK, _CTOK)]], g_v, sem)
        d0.wait()
        d1.wait()
        for r in range(_CTOK):
            def add16(ci, carry2, r=r):
                sl = pl.ds(ci * 16, 16)
                acc_v[r, sl] = acc_v[r, sl] + g_v[r, sl]
                return carry2
            lax.fori_loop(0, D // 16, add16, 0, unroll=8)
        pltpu.sync_copy(acc_v, out_hbm.at[pl.ds(base + c * _CTOK, _CTOK)])
        return carry

    lax.fori_loop(0, _NCHC, chunk, 0)


def _combine_rows(partial, p0, p1):
    mesh = plsc.VectorSubcoreMesh(core_axis_name="c", subcore_axis_name="s",
                                  num_cores=NC, num_subcores=NS)
    fn = pl.kernel(
        _combine_body,
        out_type=jax.ShapeDtypeStruct((T, D), jnp.float32),
        mesh=mesh,
        scratch_types=[
            pltpu.VMEM((_TPW,), jnp.int32),
            pltpu.VMEM((_TPW,), jnp.int32),
            pltpu.VMEM((_CTOK, D), jnp.float32),
            pltpu.VMEM((_CTOK, D), jnp.float32),
            pltpu.SemaphoreType.DMA,
        ],
    )
    return fn(partial, p0, p1)


def kernel(x, W_router, w1, w2):
    b, s, d = x.shape
    x_flat = x.reshape(T, D)
    pos0, pos1, cw0, cw1, blke, z, x_packed = _router_call(x_flat, W_router)
    slot_tok_col, slot_cw = _slots_call(pos0.reshape(1, T), pos1.reshape(1, T),
                                        cw0.reshape(1, T), cw1.reshape(1, T))
    slot_tok = slot_tok_col.reshape(S)
    xs_packed = _gather_rows(x_packed, slot_tok)
    partial = _mlp_call(blke.reshape(NB), xs_packed, w1, w2, slot_cw)
    out_flat = _combine_rows(partial, pos0.reshape(T), pos1.reshape(T))
    return out_flat.reshape(b, s, d), z.reshape(())
